# baseline (speedup 1.0000x reference)
import functools

import jax
import jax.numpy as jnp
from jax import lax
from jax.experimental import pallas as pl
from jax.experimental.pallas import tpu as pltpu
from jax.experimental.pallas import tpu_sc as plsc

NUM_EXPERTS = 64
TOP_K = 8
D_MODEL = 4096
N_TOKENS = 16384
TOKEN_BLOCK = 1024
N_CHUNKS = 2
CHUNK = N_TOKENS // N_CHUNKS

_NUM_WORKERS = 32
_LANES = 16


def _tc_body(x_ref, wt_ref, logits_t_ref, probs_ref):
    logits = jnp.dot(x_ref[...], wt_ref[...], preferred_element_type=jnp.float32)
    logits_t_ref[...] = logits.T
    m = jnp.max(logits, axis=-1, keepdims=True)
    e = jnp.exp(logits - m)
    probs_ref[...] = e / jnp.sum(e, axis=-1, keepdims=True)


def _tc_call(x2, wt, chunk):
    blocks_per_chunk = CHUNK // TOKEN_BLOCK
    base = chunk * blocks_per_chunk
    return pl.pallas_call(
        _tc_body,
        grid=(blocks_per_chunk,),
        in_specs=[
            pl.BlockSpec((TOKEN_BLOCK, D_MODEL), lambda i: (base + i, 0)),
            pl.BlockSpec((D_MODEL, NUM_EXPERTS), lambda i: (0, 0)),
        ],
        out_specs=[
            pl.BlockSpec((NUM_EXPERTS, TOKEN_BLOCK), lambda i: (0, i)),
            pl.BlockSpec((TOKEN_BLOCK, NUM_EXPERTS), lambda i: (i, 0)),
        ],
        out_shape=[
            jax.ShapeDtypeStruct((NUM_EXPERTS, CHUNK), jnp.float32),
            jax.ShapeDtypeStruct((CHUNK, NUM_EXPERTS), jnp.float32),
        ],
        compiler_params=pltpu.CompilerParams(
            dimension_semantics=("arbitrary",),
        ),
    )(x2, wt)


def _sc_topk(tok_per_w, logits_t_hbm, wts_t_hbm, idx_t_hbm, lg_v, wts_v, idx_v):
    wid = lax.axis_index("s") * 2 + lax.axis_index("c")
    base = wid * tok_per_w
    pltpu.sync_copy(logits_t_hbm.at[:, pl.ds(base, tok_per_w)], lg_v)

    def group_body(g, carry):
        g16 = g * _LANES
        neg_inf = jnp.full((_LANES,), -jnp.inf, jnp.float32)
        zero_i = jnp.zeros((_LANES,), jnp.int32)
        tv = [neg_inf] * TOP_K
        ti = [zero_i] * TOP_K
        for e in range(NUM_EXPERTS):
            col = jnp.full((_LANES,), e, jnp.int32)
            v = lg_v[e, pl.ds(g16, _LANES)]
            pred = [v > tv[j] for j in range(TOP_K)]
            ntv, nti = [], []
            for j in range(TOP_K):
                if j == 0:
                    shift_v, shift_i = v, col
                else:
                    shift_v = jnp.where(pred[j - 1], tv[j - 1], v)
                    shift_i = jnp.where(pred[j - 1], ti[j - 1], col)
                ntv.append(jnp.where(pred[j], shift_v, tv[j]))
                nti.append(jnp.where(pred[j], shift_i, ti[j]))
            tv, ti = ntv, nti
        ew = [jnp.exp(tv[j] - tv[0]) for j in range(TOP_K)]
        s = ew[0]
        for j in range(1, TOP_K):
            s = s + ew[j]
        inv = 1.0 / s
        for j in range(TOP_K):
            wts_v[j, pl.ds(g16, _LANES)] = ew[j] * inv
            idx_v[j, pl.ds(g16, _LANES)] = ti[j]
        return carry

    lax.fori_loop(0, tok_per_w // _LANES, group_body, 0)

    pltpu.sync_copy(wts_v, wts_t_hbm.at[:, pl.ds(base, tok_per_w)])
    pltpu.sync_copy(idx_v, idx_t_hbm.at[:, pl.ds(base, tok_per_w)])


def _make_sc_call(n_tok):
    tok_per_w = n_tok // _NUM_WORKERS
    return functools.partial(
        pl.kernel,
        out_type=[
            jax.ShapeDtypeStruct((TOP_K, n_tok), jnp.float32),
            jax.ShapeDtypeStruct((TOP_K, n_tok), jnp.int32),
        ],
        mesh=plsc.VectorSubcoreMesh(core_axis_name="c", subcore_axis_name="s"),
        scratch_types=[
            pltpu.VMEM((NUM_EXPERTS, tok_per_w), jnp.float32),
            pltpu.VMEM((TOP_K, tok_per_w), jnp.float32),
            pltpu.VMEM((TOP_K, tok_per_w), jnp.int32),
        ],
    )(functools.partial(_sc_topk, tok_per_w))


_sc_call = _make_sc_call(CHUNK)


@jax.jit
def kernel(x, W_gate):
    b, s, d = x.shape
    n = b * s
    x2 = x.reshape(n, d)
    wt = W_gate.T
    probs_parts, wts_parts, idx_parts = [], [], []
    for c in range(N_CHUNKS):
        logits_t, probs = _tc_call(x2, wt, c)
        wts_t, idx_t = _sc_call(logits_t)
        probs_parts.append(probs)
        wts_parts.append(wts_t)
        idx_parts.append(idx_t)
    probs = jnp.concatenate(probs_parts, axis=0)
    wts_t = jnp.concatenate(wts_parts, axis=1)
    idx_t = jnp.concatenate(idx_parts, axis=1)
    return (
        wts_t.T.reshape(b, s, TOP_K),
        idx_t.T.reshape(b, s, TOP_K),
        probs.reshape(b, s, NUM_EXPERTS),
    )


# speedup vs baseline: 1.0288x; 1.0288x over previous
import functools

import jax
import jax.numpy as jnp
from jax import lax
from jax.experimental import pallas as pl
from jax.experimental.pallas import tpu as pltpu
from jax.experimental.pallas import tpu_sc as plsc

NUM_EXPERTS = 64
TOP_K = 8
D_MODEL = 4096
N_TOKENS = 16384
TOKEN_BLOCK = 1024
N_CHUNKS = 2
CHUNK = N_TOKENS // N_CHUNKS

_NUM_WORKERS = 32
_LANES = 16


def _tc_body(x_ref, w_ref, logits_t_ref, probs_ref):
    logits = lax.dot_general(
        x_ref[...], w_ref[...],
        (((1,), (1,)), ((), ())),
        preferred_element_type=jnp.float32,
    )
    logits_t_ref[...] = logits.T
    m = jnp.max(logits, axis=-1, keepdims=True)
    e = jnp.exp(logits - m)
    probs_ref[...] = e / jnp.sum(e, axis=-1, keepdims=True)


def _tc_call(x2, w, chunk):
    blocks_per_chunk = CHUNK // TOKEN_BLOCK
    base = chunk * blocks_per_chunk
    return pl.pallas_call(
        _tc_body,
        grid=(blocks_per_chunk,),
        in_specs=[
            pl.BlockSpec((TOKEN_BLOCK, D_MODEL), lambda i: (base + i, 0)),
            pl.BlockSpec((NUM_EXPERTS, D_MODEL), lambda i: (0, 0)),
        ],
        out_specs=[
            pl.BlockSpec((NUM_EXPERTS, TOKEN_BLOCK), lambda i: (0, i)),
            pl.BlockSpec((TOKEN_BLOCK, NUM_EXPERTS), lambda i: (i, 0)),
        ],
        out_shape=[
            jax.ShapeDtypeStruct((NUM_EXPERTS, CHUNK), jnp.float32),
            jax.ShapeDtypeStruct((CHUNK, NUM_EXPERTS), jnp.float32),
        ],
        compiler_params=pltpu.CompilerParams(
            dimension_semantics=("arbitrary",),
        ),
    )(x2, w)


def _sc_topk(tok_per_w, logits_t_hbm, wts_t_hbm, idx_t_hbm, lg_v, wts_v, idx_v):
    wid = lax.axis_index("s") * 2 + lax.axis_index("c")
    base = wid * tok_per_w
    pltpu.sync_copy(logits_t_hbm.at[:, pl.ds(base, tok_per_w)], lg_v)

    def group_body(g, carry):
        g16 = g * _LANES
        neg_inf = jnp.full((_LANES,), -jnp.inf, jnp.float32)
        zero_i = jnp.zeros((_LANES,), jnp.int32)
        tv = [neg_inf] * TOP_K
        ti = [zero_i] * TOP_K
        for e in range(NUM_EXPERTS):
            col = jnp.full((_LANES,), e, jnp.int32)
            v = lg_v[e, pl.ds(g16, _LANES)]
            pred = [v > tv[j] for j in range(TOP_K)]
            ntv, nti = [], []
            for j in range(TOP_K):
                if j == 0:
                    shift_v, shift_i = v, col
                else:
                    shift_v = jnp.where(pred[j - 1], tv[j - 1], v)
                    shift_i = jnp.where(pred[j - 1], ti[j - 1], col)
                ntv.append(jnp.where(pred[j], shift_v, tv[j]))
                nti.append(jnp.where(pred[j], shift_i, ti[j]))
            tv, ti = ntv, nti
        ew = [jnp.exp(tv[j] - tv[0]) for j in range(TOP_K)]
        s = ew[0]
        for j in range(1, TOP_K):
            s = s + ew[j]
        inv = 1.0 / s
        for j in range(TOP_K):
            wts_v[j, pl.ds(g16, _LANES)] = ew[j] * inv
            idx_v[j, pl.ds(g16, _LANES)] = ti[j]
        return carry

    lax.fori_loop(0, tok_per_w // _LANES, group_body, 0)

    pltpu.sync_copy(wts_v, wts_t_hbm.at[:, pl.ds(base, tok_per_w)])
    pltpu.sync_copy(idx_v, idx_t_hbm.at[:, pl.ds(base, tok_per_w)])


def _make_sc_call(n_tok):
    tok_per_w = n_tok // _NUM_WORKERS
    return functools.partial(
        pl.kernel,
        out_type=[
            jax.ShapeDtypeStruct((TOP_K, n_tok), jnp.float32),
            jax.ShapeDtypeStruct((TOP_K, n_tok), jnp.int32),
        ],
        mesh=plsc.VectorSubcoreMesh(core_axis_name="c", subcore_axis_name="s"),
        scratch_types=[
            pltpu.VMEM((NUM_EXPERTS, tok_per_w), jnp.float32),
            pltpu.VMEM((TOP_K, tok_per_w), jnp.float32),
            pltpu.VMEM((TOP_K, tok_per_w), jnp.int32),
        ],
    )(functools.partial(_sc_topk, tok_per_w))


_sc_call = _make_sc_call(CHUNK)


@jax.jit
def kernel(x, W_gate):
    b, s, d = x.shape
    n = b * s
    x2 = x.reshape(n, d)
    probs_parts, wts_parts, idx_parts = [], [], []
    for c in range(N_CHUNKS):
        logits_t, probs = _tc_call(x2, W_gate, c)
        wts_t, idx_t = _sc_call(logits_t)
        probs_parts.append(probs)
        wts_parts.append(wts_t)
        idx_parts.append(idx_t)
    probs = jnp.concatenate(probs_parts, axis=0)
    wts_t = jnp.concatenate(wts_parts, axis=1)
    idx_t = jnp.concatenate(idx_parts, axis=1)
    return (
        wts_t.T.reshape(b, s, TOP_K),
        idx_t.T.reshape(b, s, TOP_K),
        probs.reshape(b, s, NUM_EXPERTS),
    )


# speedup vs baseline: 1.0674x; 1.0375x over previous
import functools

import jax
import jax.numpy as jnp
from jax import lax
from jax.experimental import pallas as pl
from jax.experimental.pallas import tpu as pltpu
from jax.experimental.pallas import tpu_sc as plsc

NUM_EXPERTS = 64
TOP_K = 8
D_MODEL = 4096
N_TOKENS = 16384
TOKEN_BLOCK = 1024
N_CHUNKS = 2
CHUNK = N_TOKENS // N_CHUNKS

_NUM_WORKERS = 32
_LANES = 16


def _tc_body(x_ref, w_ref, logits_t_ref, probs_t_ref):
    logits = lax.dot_general(
        x_ref[...], w_ref[...],
        (((1,), (1,)), ((), ())),
        preferred_element_type=jnp.float32,
    )
    logits_t_ref[...] = logits.T
    m = jnp.max(logits, axis=-1, keepdims=True)
    e = jnp.exp(logits - m)
    probs_t_ref[...] = (e / jnp.sum(e, axis=-1, keepdims=True)).T


def _tc_call(x2, w, chunk):
    blocks_per_chunk = CHUNK // TOKEN_BLOCK
    base = chunk * blocks_per_chunk
    return pl.pallas_call(
        _tc_body,
        grid=(blocks_per_chunk,),
        in_specs=[
            pl.BlockSpec((TOKEN_BLOCK, D_MODEL), lambda i: (base + i, 0)),
            pl.BlockSpec((NUM_EXPERTS, D_MODEL), lambda i: (0, 0)),
        ],
        out_specs=[
            pl.BlockSpec((NUM_EXPERTS, TOKEN_BLOCK), lambda i: (0, i)),
            pl.BlockSpec((NUM_EXPERTS, TOKEN_BLOCK), lambda i: (0, i)),
        ],
        out_shape=[
            jax.ShapeDtypeStruct((NUM_EXPERTS, CHUNK), jnp.float32),
            jax.ShapeDtypeStruct((NUM_EXPERTS, CHUNK), jnp.float32),
        ],
        compiler_params=pltpu.CompilerParams(
            dimension_semantics=("arbitrary",),
        ),
    )(x2, w)


def _sc_topk(tok_per_w, logits_t_hbm, wts_t_hbm, idx_t_hbm, lg_v, wts_v, idx_v):
    wid = lax.axis_index("s") * 2 + lax.axis_index("c")
    base = wid * tok_per_w
    pltpu.sync_copy(logits_t_hbm.at[:, pl.ds(base, tok_per_w)], lg_v)

    def group_body(g, carry):
        g16 = g * _LANES
        neg_inf = jnp.full((_LANES,), -jnp.inf, jnp.float32)
        zero_i = jnp.zeros((_LANES,), jnp.int32)
        tv = [neg_inf] * TOP_K
        ti = [zero_i] * TOP_K
        for e in range(NUM_EXPERTS):
            col = jnp.full((_LANES,), e, jnp.int32)
            v = lg_v[e, pl.ds(g16, _LANES)]
            pred = [v > tv[j] for j in range(TOP_K)]
            ntv, nti = [], []
            for j in range(TOP_K):
                if j == 0:
                    shift_v, shift_i = v, col
                else:
                    shift_v = jnp.where(pred[j - 1], tv[j - 1], v)
                    shift_i = jnp.where(pred[j - 1], ti[j - 1], col)
                ntv.append(jnp.where(pred[j], shift_v, tv[j]))
                nti.append(jnp.where(pred[j], shift_i, ti[j]))
            tv, ti = ntv, nti
        ew = [jnp.exp(tv[j] - tv[0]) for j in range(TOP_K)]
        s = ew[0]
        for j in range(1, TOP_K):
            s = s + ew[j]
        inv = 1.0 / s
        for j in range(TOP_K):
            wts_v[j, pl.ds(g16, _LANES)] = ew[j] * inv
            idx_v[j, pl.ds(g16, _LANES)] = ti[j]
        return carry

    lax.fori_loop(0, tok_per_w // _LANES, group_body, 0)

    pltpu.sync_copy(wts_v, wts_t_hbm.at[:, pl.ds(base, tok_per_w)])
    pltpu.sync_copy(idx_v, idx_t_hbm.at[:, pl.ds(base, tok_per_w)])


def _make_sc_call(n_tok):
    tok_per_w = n_tok // _NUM_WORKERS
    return functools.partial(
        pl.kernel,
        out_type=[
            jax.ShapeDtypeStruct((TOP_K, n_tok), jnp.float32),
            jax.ShapeDtypeStruct((TOP_K, n_tok), jnp.int32),
        ],
        mesh=plsc.VectorSubcoreMesh(core_axis_name="c", subcore_axis_name="s"),
        scratch_types=[
            pltpu.VMEM((NUM_EXPERTS, tok_per_w), jnp.float32),
            pltpu.VMEM((TOP_K, tok_per_w), jnp.float32),
            pltpu.VMEM((TOP_K, tok_per_w), jnp.int32),
        ],
    )(functools.partial(_sc_topk, tok_per_w))


_sc_call = _make_sc_call(CHUNK)


@jax.jit
def kernel(x, W_gate):
    b, s, d = x.shape
    n = b * s
    x2 = x.reshape(n, d)
    probs_parts, wts_parts, idx_parts = [], [], []
    for c in range(N_CHUNKS):
        logits_t, probs = _tc_call(x2, W_gate, c)
        wts_t, idx_t = _sc_call(logits_t)
        probs_parts.append(probs)
        wts_parts.append(wts_t)
        idx_parts.append(idx_t)
    probs = jnp.stack(probs_parts, axis=0).transpose(0, 2, 1)
    wts = jnp.stack(wts_parts, axis=0).transpose(0, 2, 1)
    idx = jnp.stack(idx_parts, axis=0).transpose(0, 2, 1)
    return (wts, idx, probs)
